# baseline (speedup 1.0000x reference)
import jax
import jax.numpy as jnp
from jax.experimental import pallas as pl
from jax.experimental.pallas import tpu as pltpu

_D = 128
_H = 8
_HD = 16
_M = 65536
_BLK = 4096
_NB = _M // _BLK
_SCALE = 0.25
_NEG = -1e30

_PREC = jax.lax.Precision.HIGHEST


def _dot(a, b, dims):
    return jax.lax.dot_general(a, b, (dims, ((), ())),
                               preferred_element_type=jnp.float32,
                               precision=_PREC)


def _head_mask():
    lane_head = jax.lax.broadcasted_iota(jnp.int32, (_H, _D), 1) // _HD
    row = jax.lax.broadcasted_iota(jnp.int32, (_H, _D), 0)
    return (lane_head == row).astype(jnp.float32)


def _wm_body(q_ref, x_ref, w_in_ref, b_in_ref, wo_ref, bo_ref,
             att_ref, attn_ref,
             st_w, m_scr, l_scr, t_scr, sT_scr):
    i = pl.program_id(0)

    @pl.when(i == 0)
    def _init():
        wq = w_in_ref[0:_D, :]
        qp = _dot(q_ref[...], wq, ((1,), (1,))) + b_in_ref[:, 0:_D]
        q8 = _head_mask() * qp
        wk = w_in_ref[_D:2 * _D, :]
        st_w[...] = _dot(q8, wk, ((1,), (0,))) * _SCALE
        m_scr[...] = jnp.full((_H, _D), _NEG, jnp.float32)
        l_scr[...] = jnp.zeros((_H, _D), jnp.float32)
        t_scr[...] = jnp.zeros((_H, _D), jnp.float32)

    x = x_ref[...]
    s_blk = _dot(x, st_w[...], ((1,), (1,)))
    s_t = jnp.transpose(s_blk)
    sT_scr[:, pl.ds(i * _BLK, _BLK)] = s_t

    m_old = m_scr[:, 0:1]
    m_new = jnp.maximum(m_old, jnp.max(s_t, axis=1, keepdims=True))
    alpha = jnp.exp(m_old - m_new)
    p = jnp.exp(s_t - m_new)
    l_new = alpha * l_scr[:, 0:1] + jnp.sum(p, axis=1, keepdims=True)
    t_scr[...] = alpha * t_scr[...] + _dot(p, x, ((1,), (0,)))
    m_scr[...] = jnp.broadcast_to(m_new, (_H, _D))
    l_scr[...] = jnp.broadcast_to(l_new, (_H, _D))

    @pl.when(i == _NB - 1)
    def _fin():
        l_f = l_scr[:, 0:1]
        m_f = m_scr[:, 0:1]
        u = t_scr[...] / l_f
        wv = w_in_ref[2 * _D:3 * _D, :]
        proj = _dot(u, wv, ((1,), (1,)))
        out_row = (jnp.sum(proj * _head_mask(), axis=0, keepdims=True)
                   + b_in_ref[:, 2 * _D:3 * _D])
        att_ref[...] = _dot(out_row, wo_ref[...], ((1,), (1,))) + bo_ref[...]

        w = jnp.exp(sT_scr[...] - m_f) * (1.0 / l_f)
        attn_ref[...] = jnp.sum(w, axis=0, keepdims=True) * (1.0 / _H)


def _wm_call(query, working_buffer, in_proj_weight, in_proj_bias_row,
             out_proj_weight, out_proj_bias_row):
    grid = (_NB,)
    const = lambda i: (0, 0)
    return pl.pallas_call(
        _wm_body,
        grid=grid,
        in_specs=[
            pl.BlockSpec((1, _D), const),
            pl.BlockSpec((_BLK, _D), lambda i: (i, 0)),
            pl.BlockSpec((3 * _D, _D), const),
            pl.BlockSpec((1, 3 * _D), const),
            pl.BlockSpec((_D, _D), const),
            pl.BlockSpec((1, _D), const),
        ],
        out_specs=[
            pl.BlockSpec((1, _D), const),
            pl.BlockSpec((1, _M), const),
        ],
        out_shape=[
            jax.ShapeDtypeStruct((1, _D), jnp.float32),
            jax.ShapeDtypeStruct((1, _M), jnp.float32),
        ],
        scratch_shapes=[
            pltpu.VMEM((_H, _D), jnp.float32),
            pltpu.VMEM((_H, _D), jnp.float32),
            pltpu.VMEM((_H, _D), jnp.float32),
            pltpu.VMEM((_H, _D), jnp.float32),
            pltpu.VMEM((_H, _M), jnp.float32),
        ],
        compiler_params=pltpu.CompilerParams(
            dimension_semantics=("arbitrary",),
        ),
    )(query, working_buffer, in_proj_weight, in_proj_bias_row,
      out_proj_weight, out_proj_bias_row)


def kernel(query, working_buffer, in_proj_weight, in_proj_bias,
           out_proj_weight, out_proj_bias):
    b_in = in_proj_bias.reshape(1, 3 * _D)
    b_out = out_proj_bias.reshape(1, _D)
    attended, attn_row = _wm_call(query, working_buffer, in_proj_weight,
                                  b_in, out_proj_weight, b_out)
    return attended, attn_row.reshape(1, 1, _M)


# speedup vs baseline: 2.7382x; 2.7382x over previous
import jax
import jax.numpy as jnp
from jax.experimental import pallas as pl
from jax.experimental.pallas import tpu as pltpu

_D = 128
_H = 8
_HD = 16
_M = 65536
_BLK = 4096
_NB = _M // _BLK
_SCALE = 0.25
_NEG = -1e30

_PREC = jax.lax.Precision.HIGHEST


def _dot(a, b, dims):
    return jax.lax.dot_general(a, b, (dims, ((), ())),
                               preferred_element_type=jnp.float32,
                               precision=_PREC)


def _head_mask():
    lane_head = jax.lax.broadcasted_iota(jnp.int32, (_H, _D), 1) // _HD
    row = jax.lax.broadcasted_iota(jnp.int32, (_H, _D), 0)
    return (lane_head == row).astype(jnp.float32)


def _wm_body(q_ref, x_ref, w_in_ref, b_in_ref, wo_ref, bo_ref,
             att_ref, attn_ref,
             st_w, m_scr, l_scr, t_scr, sT_scr):
    i = pl.program_id(0)

    @pl.when(i == 0)
    def _init():
        wq = w_in_ref[0:_D, :]
        qp = _dot(q_ref[...], wq, ((1,), (1,))) + b_in_ref[:, 0:_D]
        q8 = _head_mask() * qp
        wk = w_in_ref[_D:2 * _D, :]
        st_w[...] = _dot(q8, wk, ((1,), (0,))) * _SCALE
        m_scr[...] = jnp.full((_H, _D), _NEG, jnp.float32)
        l_scr[...] = jnp.zeros((_H, _D), jnp.float32)
        t_scr[...] = jnp.zeros((_H, _D), jnp.float32)

    xb = x_ref[...].astype(jnp.bfloat16)
    stb = st_w[...].astype(jnp.bfloat16)
    s_blk = jax.lax.dot_general(xb, stb, (((1,), (1,)), ((), ())),
                                preferred_element_type=jnp.float32)
    s_t = jnp.transpose(s_blk)
    sT_scr[:, pl.ds(i * _BLK, _BLK)] = s_t

    m_old = m_scr[:, 0:1]
    m_new = jnp.maximum(m_old, jnp.max(s_t, axis=1, keepdims=True))
    alpha = jnp.exp(m_old - m_new)
    p = jnp.exp(s_t - m_new)
    l_new = alpha * l_scr[:, 0:1] + jnp.sum(p, axis=1, keepdims=True)
    pb = p.astype(jnp.bfloat16)
    t_scr[...] = alpha * t_scr[...] + jax.lax.dot_general(
        pb, xb, (((1,), (0,)), ((), ())),
        preferred_element_type=jnp.float32)
    m_scr[...] = jnp.broadcast_to(m_new, (_H, _D))
    l_scr[...] = jnp.broadcast_to(l_new, (_H, _D))

    @pl.when(i == _NB - 1)
    def _fin():
        l_f = l_scr[:, 0:1]
        m_f = m_scr[:, 0:1]
        u = t_scr[...] / l_f
        wv = w_in_ref[2 * _D:3 * _D, :]
        proj = _dot(u, wv, ((1,), (1,)))
        out_row = (jnp.sum(proj * _head_mask(), axis=0, keepdims=True)
                   + b_in_ref[:, 2 * _D:3 * _D])
        att_ref[...] = _dot(out_row, wo_ref[...], ((1,), (1,))) + bo_ref[...]

        w = jnp.exp(sT_scr[...] - m_f) * (1.0 / l_f)
        attn_ref[...] = jnp.sum(w, axis=0, keepdims=True) * (1.0 / _H)


def _wm_call(query, working_buffer, in_proj_weight, in_proj_bias_row,
             out_proj_weight, out_proj_bias_row):
    grid = (_NB,)
    const = lambda i: (0, 0)
    return pl.pallas_call(
        _wm_body,
        grid=grid,
        in_specs=[
            pl.BlockSpec((1, _D), const),
            pl.BlockSpec((_BLK, _D), lambda i: (i, 0)),
            pl.BlockSpec((3 * _D, _D), const),
            pl.BlockSpec((1, 3 * _D), const),
            pl.BlockSpec((_D, _D), const),
            pl.BlockSpec((1, _D), const),
        ],
        out_specs=[
            pl.BlockSpec((1, _D), const),
            pl.BlockSpec((1, _M), const),
        ],
        out_shape=[
            jax.ShapeDtypeStruct((1, _D), jnp.float32),
            jax.ShapeDtypeStruct((1, _M), jnp.float32),
        ],
        scratch_shapes=[
            pltpu.VMEM((_H, _D), jnp.float32),
            pltpu.VMEM((_H, _D), jnp.float32),
            pltpu.VMEM((_H, _D), jnp.float32),
            pltpu.VMEM((_H, _D), jnp.float32),
            pltpu.VMEM((_H, _M), jnp.float32),
        ],
        compiler_params=pltpu.CompilerParams(
            dimension_semantics=("arbitrary",),
        ),
    )(query, working_buffer, in_proj_weight, in_proj_bias_row,
      out_proj_weight, out_proj_bias_row)


def kernel(query, working_buffer, in_proj_weight, in_proj_bias,
           out_proj_weight, out_proj_bias):
    b_in = in_proj_bias.reshape(1, 3 * _D)
    b_out = out_proj_bias.reshape(1, _D)
    attended, attn_row = _wm_call(query, working_buffer, in_proj_weight,
                                  b_in, out_proj_weight, b_out)
    return attended, attn_row.reshape(1, 1, _M)


# speedup vs baseline: 3.1964x; 1.1674x over previous
import jax
import jax.numpy as jnp
from jax.experimental import pallas as pl
from jax.experimental.pallas import tpu as pltpu

_D = 128
_H = 8
_HD = 16
_M = 65536
_BLK = 4096
_NB = _M // _BLK
_SCALE = 0.25
_NEG = -1e30

_PREC = jax.lax.Precision.HIGHEST


def _dot(a, b, dims):
    return jax.lax.dot_general(a, b, (dims, ((), ())),
                               preferred_element_type=jnp.float32,
                               precision=_PREC)


def _head_mask():
    lane_head = jax.lax.broadcasted_iota(jnp.int32, (_H, _D), 1) // _HD
    row = jax.lax.broadcasted_iota(jnp.int32, (_H, _D), 0)
    return (lane_head == row).astype(jnp.float32)


def _wm_body(q_ref, x_ref, w_in_ref, b_in_ref, wo_ref, bo_ref,
             att_ref, attn_ref,
             st_w, m_scr, l_scr, t_scr, sT_scr):
    i = pl.program_id(0)

    @pl.when(i == 0)
    def _init():
        wq = w_in_ref[0:_D, :]
        qp = _dot(q_ref[...], wq, ((1,), (1,))) + b_in_ref[:, 0:_D]
        q8 = _head_mask() * qp
        wk = w_in_ref[_D:2 * _D, :]
        st_w[...] = _dot(q8, wk, ((1,), (0,))) * _SCALE
        m_scr[...] = jnp.full((_H, _D), _NEG, jnp.float32)
        l_scr[...] = jnp.zeros((_H, _D), jnp.float32)
        t_scr[...] = jnp.zeros((_H, _D), jnp.float32)

    xb = x_ref[...].astype(jnp.bfloat16)
    stb = st_w[...].astype(jnp.bfloat16)
    s_t = jax.lax.dot_general(stb, xb, (((1,), (1,)), ((), ())),
                              preferred_element_type=jnp.float32)
    sT_scr[:, pl.ds(i * _BLK, _BLK)] = s_t

    m_old = m_scr[:, 0:1]
    m_new = jnp.maximum(m_old, jnp.max(s_t, axis=1, keepdims=True))
    alpha = jnp.exp(m_old - m_new)
    p = jnp.exp(s_t - m_new)
    l_new = alpha * l_scr[:, 0:1] + jnp.sum(p, axis=1, keepdims=True)
    pb = p.astype(jnp.bfloat16)
    t_scr[...] = alpha * t_scr[...] + jax.lax.dot_general(
        pb, xb, (((1,), (0,)), ((), ())),
        preferred_element_type=jnp.float32)
    m_scr[...] = jnp.broadcast_to(m_new, (_H, _D))
    l_scr[...] = jnp.broadcast_to(l_new, (_H, _D))

    @pl.when(i == _NB - 1)
    def _fin():
        l_f = l_scr[:, 0:1]
        m_f = m_scr[:, 0:1]
        u = t_scr[...] / l_f
        wv = w_in_ref[2 * _D:3 * _D, :]
        proj = _dot(u, wv, ((1,), (1,)))
        out_row = (jnp.sum(proj * _head_mask(), axis=0, keepdims=True)
                   + b_in_ref[:, 2 * _D:3 * _D])
        att_ref[...] = _dot(out_row, wo_ref[...], ((1,), (1,))) + bo_ref[...]

        w = jnp.exp(sT_scr[...] - m_f) * (1.0 / l_f)
        attn_ref[...] = jnp.sum(w, axis=0, keepdims=True) * (1.0 / _H)


def _wm_call(query, working_buffer, in_proj_weight, in_proj_bias_row,
             out_proj_weight, out_proj_bias_row):
    grid = (_NB,)
    const = lambda i: (0, 0)
    return pl.pallas_call(
        _wm_body,
        grid=grid,
        in_specs=[
            pl.BlockSpec((1, _D), const),
            pl.BlockSpec((_BLK, _D), lambda i: (i, 0)),
            pl.BlockSpec((3 * _D, _D), const),
            pl.BlockSpec((1, 3 * _D), const),
            pl.BlockSpec((_D, _D), const),
            pl.BlockSpec((1, _D), const),
        ],
        out_specs=[
            pl.BlockSpec((1, _D), const),
            pl.BlockSpec((1, _M), const),
        ],
        out_shape=[
            jax.ShapeDtypeStruct((1, _D), jnp.float32),
            jax.ShapeDtypeStruct((1, _M), jnp.float32),
        ],
        scratch_shapes=[
            pltpu.VMEM((_H, _D), jnp.float32),
            pltpu.VMEM((_H, _D), jnp.float32),
            pltpu.VMEM((_H, _D), jnp.float32),
            pltpu.VMEM((_H, _D), jnp.float32),
            pltpu.VMEM((_H, _M), jnp.float32),
        ],
        compiler_params=pltpu.CompilerParams(
            dimension_semantics=("arbitrary",),
        ),
    )(query, working_buffer, in_proj_weight, in_proj_bias_row,
      out_proj_weight, out_proj_bias_row)


def kernel(query, working_buffer, in_proj_weight, in_proj_bias,
           out_proj_weight, out_proj_bias):
    b_in = in_proj_bias.reshape(1, 3 * _D)
    b_out = out_proj_bias.reshape(1, _D)
    attended, attn_row = _wm_call(query, working_buffer, in_proj_weight,
                                  b_in, out_proj_weight, b_out)
    return attended, attn_row.reshape(1, 1, _M)


# speedup vs baseline: 4.2736x; 1.3370x over previous
import jax
import jax.numpy as jnp
from jax.experimental import pallas as pl
from jax.experimental.pallas import tpu as pltpu

_D = 128
_H = 8
_HD = 16
_M = 65536
_BLK = 16384
_NB = _M // _BLK
_SCALE = 0.25
_NEG = -1e30

_PREC = jax.lax.Precision.HIGHEST


def _dot(a, b, dims):
    return jax.lax.dot_general(a, b, (dims, ((), ())),
                               preferred_element_type=jnp.float32,
                               precision=_PREC)


def _bdot(a, b, dims):
    return jax.lax.dot_general(a, b, (dims, ((), ())),
                               preferred_element_type=jnp.float32)


def _head_mask():
    lane_head = jax.lax.broadcasted_iota(jnp.int32, (_H, _D), 1) // _HD
    row = jax.lax.broadcasted_iota(jnp.int32, (_H, _D), 0)
    return (lane_head == row).astype(jnp.float32)


def _wm_body(q_ref, x_ref, w_in_ref, b_in_ref, wo_ref, bo_ref,
             att_ref, attn_ref,
             m_scr, l_scr, t_scr, sT_scr):
    i = pl.program_id(0)
    first = i == 0

    wq = w_in_ref[0:_D, :]
    qp = _dot(q_ref[...], wq, ((1,), (1,))) + b_in_ref[:, 0:_D]
    q8 = _head_mask() * qp
    wk = w_in_ref[_D:2 * _D, :]
    st = _dot(q8, wk, ((1,), (0,))) * _SCALE

    xb = x_ref[...].astype(jnp.bfloat16)
    stb = st.astype(jnp.bfloat16)
    s_t = _bdot(stb, xb, ((1,), (1,)))
    sT_scr[:, pl.ds(i * _BLK, _BLK)] = s_t

    m_old = jnp.where(first, _NEG, m_scr[0:_H, 0:1])
    l_old = jnp.where(first, 0.0, l_scr[0:_H, 0:1])
    t_old = jnp.where(first, 0.0, t_scr[...])
    m_new = jnp.maximum(m_old, jnp.max(s_t, axis=1, keepdims=True))
    alpha = jnp.exp(m_old - m_new)
    p = jnp.exp(s_t - m_new)
    l_new = alpha * l_old + jnp.sum(p, axis=1, keepdims=True)
    pb = p.astype(jnp.bfloat16)
    t_scr[...] = alpha * t_old + _bdot(pb, xb, ((1,), (0,)))
    m_scr[...] = jnp.broadcast_to(m_new, (_H, _D))
    l_scr[...] = jnp.broadcast_to(l_new, (_H, _D))

    @pl.when(i == _NB - 1)
    def _fin():
        l_f = l_scr[0:_H, 0:1]
        m_f = m_scr[0:_H, 0:1]
        u = t_scr[...] / l_f
        wv = w_in_ref[2 * _D:3 * _D, :]
        proj = _dot(u, wv, ((1,), (1,)))
        out_row = (jnp.sum(proj * _head_mask(), axis=0, keepdims=True)
                   + b_in_ref[:, 2 * _D:3 * _D])
        att_ref[...] = _dot(out_row, wo_ref[...], ((1,), (1,))) + bo_ref[...]

        w = jnp.exp(sT_scr[...] - m_f) * ((1.0 / _H) / l_f)
        ones8 = jnp.ones((1, _H), jnp.bfloat16)
        row = _bdot(ones8, w.astype(jnp.bfloat16), ((1,), (0,)))
        attn_ref[...] = row.reshape(1, 1, _M)


def _wm_call(query, working_buffer, in_proj_weight, in_proj_bias_row,
             out_proj_weight, out_proj_bias_row):
    grid = (_NB,)
    const = lambda i: (0, 0)
    return pl.pallas_call(
        _wm_body,
        grid=grid,
        in_specs=[
            pl.BlockSpec((1, _D), const),
            pl.BlockSpec((_BLK, _D), lambda i: (i, 0)),
            pl.BlockSpec((3 * _D, _D), const),
            pl.BlockSpec((1, 3 * _D), const),
            pl.BlockSpec((_D, _D), const),
            pl.BlockSpec((1, _D), const),
        ],
        out_specs=[
            pl.BlockSpec((1, _D), const),
            pl.BlockSpec((1, 1, _M), lambda i: (0, 0, 0)),
        ],
        out_shape=[
            jax.ShapeDtypeStruct((1, _D), jnp.float32),
            jax.ShapeDtypeStruct((1, 1, _M), jnp.float32),
        ],
        scratch_shapes=[
            pltpu.VMEM((_H, _D), jnp.float32),
            pltpu.VMEM((_H, _D), jnp.float32),
            pltpu.VMEM((_H, _D), jnp.float32),
            pltpu.VMEM((_H, _M), jnp.float32),
        ],
        compiler_params=pltpu.CompilerParams(
            dimension_semantics=("arbitrary",),
        ),
    )(query, working_buffer, in_proj_weight, in_proj_bias_row,
      out_proj_weight, out_proj_bias_row)


def kernel(query, working_buffer, in_proj_weight, in_proj_bias,
           out_proj_weight, out_proj_bias):
    b_in = in_proj_bias.reshape(1, 3 * _D)
    b_out = out_proj_bias.reshape(1, _D)
    return _wm_call(query, working_buffer, in_proj_weight,
                    b_in, out_proj_weight, b_out)


# speedup vs baseline: 4.5438x; 1.0632x over previous
import jax
import jax.numpy as jnp
from jax.experimental import pallas as pl
from jax.experimental.pallas import tpu as pltpu

_D = 128
_H = 8
_HD = 16
_M = 65536
_BLK = 16384
_NB = _M // _BLK
_SCALE = 0.25
_NEG = -1e30

_PREC = jax.lax.Precision.HIGHEST


def _dot(a, b, dims):
    return jax.lax.dot_general(a, b, (dims, ((), ())),
                               preferred_element_type=jnp.float32,
                               precision=_PREC)


def _bdot(a, b, dims):
    return jax.lax.dot_general(a, b, (dims, ((), ())),
                               preferred_element_type=jnp.float32)


def _head_mask():
    lane_head = jax.lax.broadcasted_iota(jnp.int32, (_H, _D), 1) // _HD
    row = jax.lax.broadcasted_iota(jnp.int32, (_H, _D), 0)
    return (lane_head == row).astype(jnp.float32)


def _wm_body(q_ref, x_ref, w_in_ref, b_in_ref, wo_ref, bo_ref,
             att_ref, attn_ref,
             st_w, m_scr, l_scr, t_scr, sT_scr):
    i = pl.program_id(0)

    @pl.when(i == 0)
    def _init():
        wq = w_in_ref[0:_D, :]
        qp = _dot(q_ref[...], wq, ((1,), (1,))) + b_in_ref[:, 0:_D]
        q8 = _head_mask() * qp
        wk = w_in_ref[_D:2 * _D, :]
        st_w[...] = _dot(q8, wk, ((1,), (0,))) * _SCALE
        m_scr[...] = jnp.full((_H, _D), _NEG, jnp.float32)
        l_scr[...] = jnp.zeros((_H, _D), jnp.float32)
        t_scr[...] = jnp.zeros((_H, _D), jnp.float32)

    xb = x_ref[...].astype(jnp.bfloat16)
    stb = st_w[...].astype(jnp.bfloat16)
    s_t = _bdot(stb, xb, ((1,), (1,)))
    sT_scr[:, pl.ds(i * _BLK, _BLK)] = s_t

    m_old = m_scr[0:_H, 0:1]
    m_new = jnp.maximum(m_old, jnp.max(s_t, axis=1, keepdims=True))
    alpha = jnp.exp(m_old - m_new)
    p = jnp.exp(s_t - m_new)
    l_new = alpha * l_scr[0:_H, 0:1] + jnp.sum(p, axis=1, keepdims=True)
    pb = p.astype(jnp.bfloat16)
    t_scr[...] = alpha * t_scr[...] + _bdot(pb, xb, ((1,), (0,)))
    m_scr[...] = jnp.broadcast_to(m_new, (_H, _D))
    l_scr[...] = jnp.broadcast_to(l_new, (_H, _D))

    @pl.when(i == _NB - 1)
    def _fin():
        l_f = l_scr[0:_H, 0:1]
        m_f = m_scr[0:_H, 0:1]
        u = t_scr[...] / l_f
        wv = w_in_ref[2 * _D:3 * _D, :]
        proj = _dot(u, wv, ((1,), (1,)))
        out_row = (jnp.sum(proj * _head_mask(), axis=0, keepdims=True)
                   + b_in_ref[:, 2 * _D:3 * _D])
        att_ref[...] = _dot(out_row, wo_ref[...], ((1,), (1,))) + bo_ref[...]

        w = jnp.exp(sT_scr[...] - m_f) * ((1.0 / _H) / l_f)
        ones8 = jnp.ones((1, _H), jnp.bfloat16)
        row = _bdot(ones8, w.astype(jnp.bfloat16), ((1,), (0,)))
        attn_ref[...] = row.reshape(1, 1, _M)


def _wm_call(query, working_buffer, in_proj_weight, in_proj_bias_row,
             out_proj_weight, out_proj_bias_row):
    grid = (_NB,)
    const = lambda i: (0, 0)
    return pl.pallas_call(
        _wm_body,
        grid=grid,
        in_specs=[
            pl.BlockSpec((1, _D), const),
            pl.BlockSpec((_BLK, _D), lambda i: (i, 0)),
            pl.BlockSpec((3 * _D, _D), const),
            pl.BlockSpec((1, 3 * _D), const),
            pl.BlockSpec((_D, _D), const),
            pl.BlockSpec((1, _D), const),
        ],
        out_specs=[
            pl.BlockSpec((1, _D), const),
            pl.BlockSpec((1, 1, _M), lambda i: (0, 0, 0)),
        ],
        out_shape=[
            jax.ShapeDtypeStruct((1, _D), jnp.float32),
            jax.ShapeDtypeStruct((1, 1, _M), jnp.float32),
        ],
        scratch_shapes=[
            pltpu.VMEM((_H, _D), jnp.float32),
            pltpu.VMEM((_H, _D), jnp.float32),
            pltpu.VMEM((_H, _D), jnp.float32),
            pltpu.VMEM((_H, _D), jnp.float32),
            pltpu.VMEM((_H, _M), jnp.float32),
        ],
        compiler_params=pltpu.CompilerParams(
            dimension_semantics=("arbitrary",),
        ),
    )(query, working_buffer, in_proj_weight, in_proj_bias_row,
      out_proj_weight, out_proj_bias_row)


def kernel(query, working_buffer, in_proj_weight, in_proj_bias,
           out_proj_weight, out_proj_bias):
    b_in = in_proj_bias.reshape(1, 3 * _D)
    b_out = out_proj_bias.reshape(1, _D)
    return _wm_call(query, working_buffer, in_proj_weight,
                    b_in, out_proj_weight, b_out)
